# baseline (speedup 1.0000x reference)
import functools

import jax
import jax.numpy as jnp
from jax import lax
from jax.experimental import pallas as pl
from jax.experimental.pallas import tpu as pltpu, tpu_sc as plsc

NUM_EMBEDDINGS = 1000000
EMBEDDING_DIM = 64
BATCH = 16384

_info = plsc.get_sparse_core_info()
_NC, _NS = _info.num_cores, _info.num_subcores
_NW = _NC * _NS
_BPW = BATCH // _NW
_CHUNK = 256


@functools.partial(
    pl.kernel,
    mesh=plsc.VectorSubcoreMesh(core_axis_name="c", subcore_axis_name="s"),
    out_type=jax.ShapeDtypeStruct((BATCH, EMBEDDING_DIM), jnp.float32),
    scratch_types=[
        pltpu.VMEM((_BPW,), jnp.int32),
        pltpu.VMEM((_BPW,), jnp.int32),
        pltpu.VMEM((_CHUNK, 2 * EMBEDDING_DIM), jnp.float32),
        pltpu.VMEM((_CHUNK, EMBEDDING_DIM), jnp.float32),
        pltpu.SemaphoreType.DMA,
    ],
    compiler_params=pltpu.CompilerParams(use_tc_tiling_on_sc=True),
)
def _gather_kernel(pidx_hbm, off_hbm, table_hbm, out_hbm,
                   pidx_v, off_v, rows_v, out_v, sem):
    wid = lax.axis_index("s") * _NC + lax.axis_index("c")
    base = wid * _BPW
    pltpu.sync_copy(pidx_hbm.at[pl.ds(base, _BPW)], pidx_v)
    pltpu.sync_copy(off_hbm.at[pl.ds(base, _BPW)], off_v)

    for c in range(_BPW // _CHUNK):
        pltpu.async_copy(
            table_hbm.at[pidx_v.at[pl.ds(c * _CHUNK, _CHUNK)]], rows_v, sem
        ).wait()

        def body(g, carry, c=c):
            offs = off_v[pl.ds(c * _CHUNK + g * 16, 16)]
            for r in range(16):
                i = g * 16 + r
                off = offs[r]
                for j in range(EMBEDDING_DIM // 16):
                    out_v[i, 16 * j:16 * (j + 1)] = (
                        rows_v[i, pl.ds(off + 16 * j, 16)])
            return carry

        lax.fori_loop(0, _CHUNK // 16, body, None)
        pltpu.sync_copy(out_v, out_hbm.at[pl.ds(base + c * _CHUNK, _CHUNK)])


def kernel(residue_type, W):
    idx = residue_type.astype(jnp.int32)
    pidx = lax.shift_right_logical(idx, 1)
    off = (idx & 1) * EMBEDDING_DIM
    table = W.reshape(NUM_EMBEDDINGS // 2, 2 * EMBEDDING_DIM)
    return _gather_kernel(pidx, off, table)


# speedup vs baseline: 1.6614x; 1.6614x over previous
import functools

import jax
import jax.numpy as jnp
from jax import lax
from jax.experimental import pallas as pl
from jax.experimental.pallas import tpu as pltpu, tpu_sc as plsc

NUM_EMBEDDINGS = 1000000
EMBEDDING_DIM = 64
BATCH = 16384

_info = plsc.get_sparse_core_info()
_NC, _NS = _info.num_cores, _info.num_subcores
_NW = _NC * _NS
_BPW = BATCH // _NW
_GRP = 16


@functools.partial(
    pl.kernel,
    mesh=plsc.VectorSubcoreMesh(core_axis_name="c", subcore_axis_name="s"),
    out_type=jax.ShapeDtypeStruct((BATCH, EMBEDDING_DIM), jnp.float32),
    scratch_types=[
        pltpu.VMEM((_BPW,), jnp.int32),
        pltpu.VMEM((_BPW, EMBEDDING_DIM), jnp.float32),
        pltpu.SemaphoreType.DMA,
    ],
    compiler_params=pltpu.CompilerParams(use_tc_tiling_on_sc=True),
)
def _gather_kernel(idx_hbm, table_hbm, out_hbm, idx_v, out_v, sem):
    wid = lax.axis_index("s") * _NC + lax.axis_index("c")
    base = wid * _BPW
    pltpu.sync_copy(idx_hbm.at[pl.ds(base, _BPW)], idx_v)

    def body(g, carry):
        rows = idx_v[pl.ds(g * _GRP, 16)]
        descs = []
        for r in range(_GRP):
            descs.append(pltpu.async_copy(
                table_hbm.at[pl.ds(rows[r], 1), :],
                out_v.at[pl.ds(g * _GRP + r, 1), :],
                sem,
            ))
        for d in descs:
            d.wait()
        return carry

    lax.fori_loop(0, _BPW // _GRP, body, None)
    pltpu.sync_copy(out_v, out_hbm.at[pl.ds(base, _BPW)])


def kernel(residue_type, W):
    idx = residue_type.astype(jnp.int32)
    return _gather_kernel(idx, W)


# speedup vs baseline: 1.7469x; 1.0515x over previous
import functools

import jax
import jax.numpy as jnp
from jax import lax
from jax.experimental import pallas as pl
from jax.experimental.pallas import tpu as pltpu, tpu_sc as plsc

NUM_EMBEDDINGS = 1000000
EMBEDDING_DIM = 64
BATCH = 16384

_info = plsc.get_sparse_core_info()
_NC, _NS = _info.num_cores, _info.num_subcores
_NW = _NC * _NS
_BPW = BATCH // _NW
_GRP = 16


@functools.partial(
    pl.kernel,
    mesh=plsc.VectorSubcoreMesh(core_axis_name="c", subcore_axis_name="s"),
    out_type=jax.ShapeDtypeStruct((BATCH, EMBEDDING_DIM), jnp.float32),
    scratch_types=[
        pltpu.VMEM((_BPW,), jnp.int32),
        pltpu.VMEM((_BPW, EMBEDDING_DIM), jnp.float32),
        pltpu.SemaphoreType.DMA,
    ],
    compiler_params=pltpu.CompilerParams(use_tc_tiling_on_sc=True),
)
def _gather_kernel(idx_hbm, table_hbm, out_hbm, idx_v, out_v, sem):
    wid = lax.axis_index("s") * _NC + lax.axis_index("c")
    base = wid * _BPW
    pltpu.sync_copy(idx_hbm.at[pl.ds(base, _BPW)], idx_v)

    def body(g, carry):
        rows = idx_v[pl.ds(g * _GRP, 16)]
        for r in range(_GRP):
            pltpu.async_copy(
                table_hbm.at[pl.ds(rows[r], 1), :],
                out_v.at[pl.ds(g * _GRP + r, 1), :],
                sem,
            )
        return carry

    lax.fori_loop(0, _BPW // _GRP, body, None)
    pltpu.make_async_copy(out_hbm.at[pl.ds(base, _BPW)], out_v, sem).wait()
    pltpu.sync_copy(out_v, out_hbm.at[pl.ds(base, _BPW)])


def kernel(residue_type, W):
    idx = residue_type.astype(jnp.int32)
    return _gather_kernel(idx, W)


# speedup vs baseline: 1.7505x; 1.0021x over previous
import functools

import jax
import jax.numpy as jnp
from jax import lax
from jax.experimental import pallas as pl
from jax.experimental.pallas import tpu as pltpu, tpu_sc as plsc

NUM_EMBEDDINGS = 1000000
EMBEDDING_DIM = 64
BATCH = 16384

_info = plsc.get_sparse_core_info()
_NC, _NS = _info.num_cores, _info.num_subcores
_NW = _NC * _NS
_BPW = BATCH // _NW
_CHUNK = 256

_RB = 2048
_GRID = (NUM_EMBEDDINGS + 2 * _RB - 1) // (2 * _RB)
_TROWS = _GRID * _RB


def _transpose_body(wt_ref, out_ref):
    x = wt_ref[...]
    lo = x[:, :_RB]
    hi = x[:, _RB:]
    out_ref[...] = jnp.concatenate([lo.T, hi.T], axis=1)


_tc_transpose = pl.pallas_call(
    _transpose_body,
    grid=(_GRID,),
    in_specs=[pl.BlockSpec((EMBEDDING_DIM, 2 * _RB), lambda k: (0, k))],
    out_specs=pl.BlockSpec((_RB, 2 * EMBEDDING_DIM), lambda k: (k, 0)),
    out_shape=jax.ShapeDtypeStruct((_TROWS, 2 * EMBEDDING_DIM), jnp.float32),
)


@functools.partial(
    pl.kernel,
    mesh=plsc.VectorSubcoreMesh(core_axis_name="c", subcore_axis_name="s"),
    out_type=jax.ShapeDtypeStruct((BATCH, EMBEDDING_DIM), jnp.float32),
    scratch_types=[
        pltpu.VMEM((_BPW,), jnp.int32),
        pltpu.VMEM((_BPW,), jnp.int32),
        pltpu.VMEM((_CHUNK, 2 * EMBEDDING_DIM), jnp.float32),
        pltpu.VMEM((_CHUNK, EMBEDDING_DIM), jnp.float32),
        pltpu.SemaphoreType.DMA,
    ],
    compiler_params=pltpu.CompilerParams(use_tc_tiling_on_sc=True),
)
def _gather_kernel(pidx_hbm, off_hbm, table_hbm, out_hbm,
                   pidx_v, off_v, rows_v, out_v, sem):
    wid = lax.axis_index("s") * _NC + lax.axis_index("c")
    base = wid * _BPW
    pltpu.sync_copy(pidx_hbm.at[pl.ds(base, _BPW)], pidx_v)
    pltpu.sync_copy(off_hbm.at[pl.ds(base, _BPW)], off_v)

    for c in range(_BPW // _CHUNK):
        pltpu.async_copy(
            table_hbm.at[pidx_v.at[pl.ds(c * _CHUNK, _CHUNK)]], rows_v, sem
        ).wait()

        def body(g, carry, c=c):
            offs = off_v[pl.ds(c * _CHUNK + g * 16, 16)]
            for r in range(16):
                i = g * 16 + r
                off = offs[r]
                for j in range(EMBEDDING_DIM // 16):
                    out_v[i, 16 * j:16 * (j + 1)] = (
                        rows_v[i, pl.ds(off + 16 * j, 16)])
            return carry

        lax.fori_loop(0, _CHUNK // 16, body, None)
        pltpu.sync_copy(out_v, out_hbm.at[pl.ds(base + c * _CHUNK, _CHUNK)])


def kernel(residue_type, W):
    idx = residue_type.astype(jnp.int32)
    pidx = lax.shift_right_logical(idx, 12) * _RB + (idx & (_RB - 1))
    off = (lax.shift_right_logical(idx, 11) & 1) * EMBEDDING_DIM
    table = _tc_transpose(W.T)
    return _gather_kernel(pidx, off, table)


# speedup vs baseline: 2.4094x; 1.3764x over previous
import functools

import jax
import jax.numpy as jnp
from jax import lax
from jax.experimental import pallas as pl
from jax.experimental.pallas import tpu as pltpu, tpu_sc as plsc

NUM_EMBEDDINGS = 1000000
EMBEDDING_DIM = 64
BATCH = 16384

_info = plsc.get_sparse_core_info()
_NC, _NS = _info.num_cores, _info.num_subcores
_NW = _NC * _NS
_BPW = BATCH // _NW
_CHUNK = 256

_RB = 8192
_GRID = (NUM_EMBEDDINGS + 2 * _RB - 1) // (2 * _RB)
_TROWS = _GRID * _RB


def _transpose_body(wt_ref, out_ref):
    x = wt_ref[...]
    lo = x[:, :_RB]
    hi = x[:, _RB:]
    out_ref[...] = jnp.concatenate([lo.T, hi.T], axis=1)


_tc_transpose = pl.pallas_call(
    _transpose_body,
    grid=(_GRID,),
    in_specs=[pl.BlockSpec((EMBEDDING_DIM, 2 * _RB), lambda k: (0, k))],
    out_specs=pl.BlockSpec((_RB, 2 * EMBEDDING_DIM), lambda k: (k, 0)),
    out_shape=jax.ShapeDtypeStruct((_TROWS, 2 * EMBEDDING_DIM), jnp.float32),
)


@functools.partial(
    pl.kernel,
    mesh=plsc.VectorSubcoreMesh(core_axis_name="c", subcore_axis_name="s"),
    out_type=jax.ShapeDtypeStruct((BATCH, EMBEDDING_DIM), jnp.float32),
    scratch_types=[
        pltpu.VMEM((_BPW,), jnp.int32),
        pltpu.VMEM((_BPW,), jnp.int32),
        pltpu.VMEM((_CHUNK, 2 * EMBEDDING_DIM), jnp.float32),
        pltpu.VMEM((_CHUNK, EMBEDDING_DIM), jnp.float32),
        pltpu.SemaphoreType.DMA,
    ],
    compiler_params=pltpu.CompilerParams(use_tc_tiling_on_sc=True),
)
def _gather_kernel(pidx_hbm, off_hbm, table_hbm, out_hbm,
                   pidx_v, off_v, rows_v, out_v, sem):
    wid = lax.axis_index("s") * _NC + lax.axis_index("c")
    base = wid * _BPW
    pltpu.sync_copy(pidx_hbm.at[pl.ds(base, _BPW)], pidx_v)
    pltpu.sync_copy(off_hbm.at[pl.ds(base, _BPW)], off_v)

    for c in range(_BPW // _CHUNK):
        pltpu.async_copy(
            table_hbm.at[pidx_v.at[pl.ds(c * _CHUNK, _CHUNK)]], rows_v, sem
        ).wait()

        def body(g, carry, c=c):
            offs = off_v[pl.ds(c * _CHUNK + g * 16, 16)]
            for r in range(16):
                i = g * 16 + r
                off = offs[r]
                for j in range(EMBEDDING_DIM // 16):
                    out_v[i, 16 * j:16 * (j + 1)] = (
                        rows_v[i, pl.ds(off + 16 * j, 16)])
            return carry

        lax.fori_loop(0, _CHUNK // 16, body, None)
        pltpu.sync_copy(out_v, out_hbm.at[pl.ds(base + c * _CHUNK, _CHUNK)])


def kernel(residue_type, W):
    idx = residue_type.astype(jnp.int32)
    pidx = lax.shift_right_logical(idx, 12) * _RB + (idx & (_RB - 1))
    off = (lax.shift_right_logical(idx, 11) & 1) * EMBEDDING_DIM
    table = _tc_transpose(W.T)
    return _gather_kernel(pidx, off, table)


# speedup vs baseline: 2.5497x; 1.0582x over previous
import functools

import jax
import jax.numpy as jnp
from jax import lax
from jax.experimental import pallas as pl
from jax.experimental.pallas import tpu as pltpu, tpu_sc as plsc

NUM_EMBEDDINGS = 1000000
EMBEDDING_DIM = 64
BATCH = 16384

_info = plsc.get_sparse_core_info()
_NC, _NS = _info.num_cores, _info.num_subcores
_NW = _NC * _NS
_BPW = BATCH // _NW
_CHUNK = 256

_RB = 16384
_GRID = (NUM_EMBEDDINGS + 2 * _RB - 1) // (2 * _RB)
_TROWS = _GRID * _RB


def _transpose_body(wt_ref, out_ref):
    x = wt_ref[...]
    lo = x[:, :_RB]
    hi = x[:, _RB:]
    out_ref[...] = jnp.concatenate([lo.T, hi.T], axis=1)


_tc_transpose = pl.pallas_call(
    _transpose_body,
    grid=(_GRID,),
    in_specs=[pl.BlockSpec((EMBEDDING_DIM, 2 * _RB), lambda k: (0, k))],
    out_specs=pl.BlockSpec((_RB, 2 * EMBEDDING_DIM), lambda k: (k, 0)),
    out_shape=jax.ShapeDtypeStruct((_TROWS, 2 * EMBEDDING_DIM), jnp.float32),
)


@functools.partial(
    pl.kernel,
    mesh=plsc.VectorSubcoreMesh(core_axis_name="c", subcore_axis_name="s"),
    out_type=jax.ShapeDtypeStruct((BATCH, EMBEDDING_DIM), jnp.float32),
    scratch_types=[
        pltpu.VMEM((_BPW,), jnp.int32),
        pltpu.VMEM((_BPW,), jnp.int32),
        pltpu.VMEM((_CHUNK, 2 * EMBEDDING_DIM), jnp.float32),
        pltpu.VMEM((_CHUNK, EMBEDDING_DIM), jnp.float32),
        pltpu.SemaphoreType.DMA,
    ],
    compiler_params=pltpu.CompilerParams(use_tc_tiling_on_sc=True),
)
def _gather_kernel(pidx_hbm, off_hbm, table_hbm, out_hbm,
                   pidx_v, off_v, rows_v, out_v, sem):
    wid = lax.axis_index("s") * _NC + lax.axis_index("c")
    base = wid * _BPW
    pltpu.sync_copy(pidx_hbm.at[pl.ds(base, _BPW)], pidx_v)
    pltpu.sync_copy(off_hbm.at[pl.ds(base, _BPW)], off_v)

    for c in range(_BPW // _CHUNK):
        pltpu.async_copy(
            table_hbm.at[pidx_v.at[pl.ds(c * _CHUNK, _CHUNK)]], rows_v, sem
        ).wait()

        def body(g, carry, c=c):
            offs = off_v[pl.ds(c * _CHUNK + g * 16, 16)]
            for r in range(16):
                i = g * 16 + r
                off = offs[r]
                for j in range(EMBEDDING_DIM // 16):
                    out_v[i, 16 * j:16 * (j + 1)] = (
                        rows_v[i, pl.ds(off + 16 * j, 16)])
            return carry

        lax.fori_loop(0, _CHUNK // 16, body, None)
        pltpu.sync_copy(out_v, out_hbm.at[pl.ds(base + c * _CHUNK, _CHUNK)])


def kernel(residue_type, W):
    idx = residue_type.astype(jnp.int32)
    pidx = (idx // (2 * _RB)) * _RB + (idx % _RB)
    off = ((idx // _RB) % 2) * EMBEDDING_DIM
    table = _tc_transpose(W.T)
    return _gather_kernel(pidx, off, table)


# speedup vs baseline: 2.5678x; 1.0071x over previous
import functools

import jax
import jax.numpy as jnp
from jax import lax
from jax.experimental import pallas as pl
from jax.experimental.pallas import tpu as pltpu, tpu_sc as plsc

NUM_EMBEDDINGS = 1000000
EMBEDDING_DIM = 64
BATCH = 16384

_info = plsc.get_sparse_core_info()
_NC, _NS = _info.num_cores, _info.num_subcores
_NW = _NC * _NS
_BPW = BATCH // _NW
_CHUNK = 128

_RB = 16384
_GRID = (NUM_EMBEDDINGS + 2 * _RB - 1) // (2 * _RB)
_TROWS = _GRID * _RB


def _transpose_body(wt_ref, out_ref):
    x = wt_ref[...]
    lo = x[:, :_RB]
    hi = x[:, _RB:]
    out_ref[...] = jnp.concatenate([lo.T, hi.T], axis=1)


_tc_transpose = pl.pallas_call(
    _transpose_body,
    grid=(_GRID,),
    in_specs=[pl.BlockSpec((EMBEDDING_DIM, 2 * _RB), lambda k: (0, k))],
    out_specs=pl.BlockSpec((_RB, 2 * EMBEDDING_DIM), lambda k: (k, 0)),
    out_shape=jax.ShapeDtypeStruct((_TROWS, 2 * EMBEDDING_DIM), jnp.float32),
)


@functools.partial(
    pl.kernel,
    mesh=plsc.VectorSubcoreMesh(core_axis_name="c", subcore_axis_name="s"),
    out_type=jax.ShapeDtypeStruct((BATCH, EMBEDDING_DIM), jnp.float32),
    scratch_types=[
        pltpu.VMEM((_BPW,), jnp.int32),
        pltpu.VMEM((_BPW,), jnp.int32),
        pltpu.VMEM((_CHUNK, 2 * EMBEDDING_DIM), jnp.float32),
        pltpu.VMEM((_CHUNK, 2 * EMBEDDING_DIM), jnp.float32),
        pltpu.VMEM((_CHUNK, EMBEDDING_DIM), jnp.float32),
        pltpu.SemaphoreType.DMA,
        pltpu.SemaphoreType.DMA,
    ],
    compiler_params=pltpu.CompilerParams(use_tc_tiling_on_sc=True),
)
def _gather_kernel(pidx_hbm, off_hbm, table_hbm, out_hbm,
                   pidx_v, off_v, rows_a, rows_b, out_v, sem_a, sem_b):
    wid = lax.axis_index("s") * _NC + lax.axis_index("c")
    base = wid * _BPW
    pltpu.sync_copy(pidx_hbm.at[pl.ds(base, _BPW)], pidx_v)
    pltpu.sync_copy(off_hbm.at[pl.ds(base, _BPW)], off_v)

    bufs = [(rows_a, sem_a), (rows_b, sem_b)]
    nch = _BPW // _CHUNK

    def start(c):
        rows_v, sem = bufs[c % 2]
        return pltpu.async_copy(
            table_hbm.at[pidx_v.at[pl.ds(c * _CHUNK, _CHUNK)]], rows_v, sem)

    descs = {0: start(0)}
    for c in range(nch):
        descs[c].wait()
        if c + 1 < nch:
            descs[c + 1] = start(c + 1)
        rows_v, _ = bufs[c % 2]

        def body(g, carry, c=c, rows_v=rows_v):
            offs = off_v[pl.ds(c * _CHUNK + g * 16, 16)]
            for r in range(16):
                i = g * 16 + r
                off = offs[r]
                for j in range(EMBEDDING_DIM // 16):
                    out_v[i, 16 * j:16 * (j + 1)] = (
                        rows_v[i, pl.ds(off + 16 * j, 16)])
            return carry

        lax.fori_loop(0, _CHUNK // 16, body, None)
        pltpu.sync_copy(out_v, out_hbm.at[pl.ds(base + c * _CHUNK, _CHUNK)])


def kernel(residue_type, W):
    idx = residue_type.astype(jnp.int32)
    pidx = (idx // (2 * _RB)) * _RB + (idx % _RB)
    off = ((idx // _RB) % 2) * EMBEDDING_DIM
    table = _tc_transpose(W.T)
    return _gather_kernel(pidx, off, table)
